# baseline (speedup 1.0000x reference)
import jax
import jax.numpy as jnp
from jax.experimental import pallas as pl

NT = 64


def _pool_body(x_ref, o_ref):
    win = x_ref.shape[2]
    o_ref[0, :, :] = jnp.sum(x_ref[0], axis=1) * (1.0 / win)


def kernel(features):
    B, N, D = features.shape
    win = N // NT
    split = 2
    nt_blk = NT // split
    x = features.reshape(B * split, nt_blk, win, D)
    out = pl.pallas_call(
        _pool_body,
        grid=(B * split,),
        in_specs=[pl.BlockSpec((1, nt_blk, win, D), lambda b: (b, 0, 0, 0))],
        out_specs=pl.BlockSpec((1, nt_blk, D), lambda b: (b, 0, 0)),
        out_shape=jax.ShapeDtypeStruct((B * split, nt_blk, D), jnp.float32),
    )(x)
    return out.reshape(B, NT, D)


# speedup vs baseline: 1.6939x; 1.6939x over previous
import jax
import jax.numpy as jnp
from jax.experimental import pallas as pl

NT = 64


def _pool_body(x_ref, o_ref):
    win = x_ref.shape[2]
    o_ref[0, :, :] = jnp.sum(x_ref[0], axis=1) * (1.0 / win)


def kernel(features):
    B, N, D = features.shape
    win = N // NT
    group = 2
    nt_blk = NT * group
    x = features.reshape(B // group, nt_blk, win, D)
    out = pl.pallas_call(
        _pool_body,
        grid=(B // group,),
        in_specs=[pl.BlockSpec((1, nt_blk, win, D), lambda b: (b, 0, 0, 0))],
        out_specs=pl.BlockSpec((1, nt_blk, D), lambda b: (b, 0, 0)),
        out_shape=jax.ShapeDtypeStruct((B // group, nt_blk, D), jnp.float32),
    )(x)
    return out.reshape(B, NT, D)


# speedup vs baseline: 1.6967x; 1.0016x over previous
import jax
import jax.numpy as jnp
from jax.experimental import pallas as pl

NT = 64


def _pool_body(x_ref, o_ref):
    win = x_ref.shape[2]
    o_ref[0, :, :] = jnp.sum(x_ref[0], axis=1) * (1.0 / win)


def kernel(features):
    B, N, D = features.shape
    win = N // NT
    group = 4
    nt_blk = NT * group
    x = features.reshape(B // group, nt_blk, win, D)
    out = pl.pallas_call(
        _pool_body,
        grid=(B // group,),
        in_specs=[pl.BlockSpec((1, nt_blk, win, D), lambda b: (b, 0, 0, 0))],
        out_specs=pl.BlockSpec((1, nt_blk, D), lambda b: (b, 0, 0)),
        out_shape=jax.ShapeDtypeStruct((B // group, nt_blk, D), jnp.float32),
    )(x)
    return out.reshape(B, NT, D)
